# baseline (speedup 1.0000x reference)
import dataclasses

import jax
import jax.numpy as jnp
from jax import lax
from jax.experimental import pallas as pl
from jax.experimental.pallas import tpu as pltpu
from jax.experimental.pallas import tpu_sc as plsc

N = 50000
NP = 53760
E = 800000
EP = 819200
FEAT = 17
W1 = 128
H = 128
K = 6
M = 12
EPS = 1e-5

BLK_E = 1280
NBLK_E = EP // 16 // BLK_E
SLICE_E = EP // 16
MROWS = 11

R2 = 8960
NR = 3

BLKN = 1680
NB = NP // BLKN

_SC_MESH = dict(core_axis_name="c", subcore_axis_name="s",
                num_cores=2, num_subcores=16)


def _make_sc_agg(width, rows_per_chunk, npass, interpret=False):
    R = rows_per_chunk
    stripe = R // 16
    mesh = plsc.VectorSubcoreMesh(**_SC_MESH)
    out_type = jax.ShapeDtypeStruct((NP, width), jnp.float32)
    scratch = (
        [pltpu.VMEM((BLK_E,), jnp.int32) for _ in range(4)]
        + [pltpu.VMEM((MROWS, 128), jnp.int32) for _ in range(2)]
        + [pltpu.VMEM((NR, 128, width), jnp.float32),
           pltpu.VMEM_SHARED((R + 16, width), jnp.float32)]
        + [pltpu.SemaphoreType.DMA] * (2 + 2 * NR)
    )

    def body(src_hbm, dst_hbm, table_hbm, agg_hbm,
             es0, ed0, es1, ed1, sbuf_m, dbuf_m, rows4, acc,
             sem_e0, sem_e1, *gssems):
        c = lax.axis_index("c")
        s = lax.axis_index("s")
        ebufs = ((es0, ed0), (es1, ed1))
        esems = (sem_e0, sem_e1)
        gsems = gssems[:NR]
        ssems = gssems[NR:]

        zero16 = jnp.zeros((16,), jnp.float32)

        def load_block(b_idx, which):
            base = s * SLICE_E + b_idx * BLK_E
            pltpu.async_copy(src_hbm.at[pl.ds(base, BLK_E)],
                             ebufs[which][0], esems[which])
            pltpu.async_copy(dst_hbm.at[pl.ds(base, BLK_E)],
                             ebufs[which][1], esems[which])

        def wait_block(which):
            pltpu.make_async_copy(src_hbm.at[pl.ds(0, BLK_E)],
                                  ebufs[which][0], esems[which]).wait()
            pltpu.make_async_copy(dst_hbm.at[pl.ds(0, BLK_E)],
                                  ebufs[which][1], esems[which]).wait()

        for p in range(npass):
            lo = (2 * p + c) * R

            @pl.loop(0, 128)
            def _(j):
                @pl.loop(0, width // 16)
                def _(q):
                    rows4[0, j, pl.ds(q * 16, 16)] = zero16
            for g in range(stripe // 128):
                pltpu.sync_copy(rows4.at[0],
                                acc.at[pl.ds(s * stripe + g * 128, 128)])
            rem_z = stripe % 128
            if rem_z:
                pltpu.sync_copy(
                    rows4.at[0].at[pl.ds(0, rem_z)],
                    acc.at[pl.ds(s * stripe + (stripe // 128) * 128, rem_z)])
            plsc.subcore_barrier()

            def scan(which, m_vec):
                sb, db = ebufs[which]

                def scan_vec(j, m):
                    dv = db[pl.ds(j * 16, 16)]
                    sv = sb[pl.ds(j * 16, 16)]
                    mask = (dv >= lo) & (dv < lo + R)
                    mi = jnp.where(mask, 1, 0).astype(jnp.int32)
                    csum = plsc.cumsum(mi)
                    pos = csum + m - 1
                    prow = lax.shift_right_logical(pos, 7)
                    pcol = lax.bitwise_and(pos, 127)
                    plsc.store_scatter(sbuf_m, [prow, pcol], sv, mask=mask)
                    plsc.store_scatter(dbuf_m, [prow, pcol], dv - lo,
                                       mask=mask)
                    return m + plsc.all_reduce_population_count(mask)

                return lax.fori_loop(0, BLK_E // 16, scan_vec, m_vec)

            def flush_full(m_vec):
                nf = lax.shift_right_logical(jnp.max(m_vec), 7)
                nq = lax.div(nf + (NR - 1), NR)

                def ring(g, _):
                    for b in range(NR):
                        r = g * NR + b
                        ok = r < nf

                        @pl.when(ok & (g > 0))
                        def _():
                            pltpu.make_async_copy(
                                rows4.at[b], acc.at[dbuf_m.at[r - NR]],
                                ssems[b]).wait()

                        @pl.when(ok)
                        def _():
                            pltpu.async_copy(table_hbm.at[sbuf_m.at[r]],
                                             rows4.at[b], gsems[b])
                    for b in range(NR):
                        r = g * NR + b
                        ok = r < nf

                        @pl.when(ok)
                        def _():
                            pltpu.make_async_copy(
                                table_hbm.at[sbuf_m.at[r]], rows4.at[b],
                                gsems[b]).wait()
                            pltpu.async_copy(rows4.at[b],
                                             acc.at[dbuf_m.at[r]],
                                             ssems[b], add=True)
                    return 0

                lax.fori_loop(0, nq, ring, 0)
                for b in range(NR):
                    @pl.when(nf > b)
                    def _():
                        pltpu.make_async_copy(rows4.at[b],
                                              acc.at[dbuf_m.at[0]],
                                              ssems[b]).wait()

                @pl.when(nf > 0)
                def _():
                    for qq in range(8):
                        sl = pl.ds(qq * 16, 16)
                        sbuf_m[0, sl] = sbuf_m[nf, sl]
                        dbuf_m[0, sl] = dbuf_m[nf, sl]
                return lax.bitwise_and(m_vec, 127)

            load_block(0, 0)

            def pair(q, m_vec):
                wait_block(0)
                load_block(2 * q + 1, 1)
                m_vec = flush_full(scan(0, m_vec))
                wait_block(1)

                @pl.when(q < NBLK_E // 2 - 1)
                def _():
                    load_block(2 * q + 2, 0)
                m_vec = flush_full(scan(1, m_vec))
                return m_vec

            m_vec = lax.fori_loop(0, NBLK_E // 2, pair,
                                  jnp.zeros((16,), jnp.int32))

            rem_vec = lax.bitwise_and(m_vec, 127)
            lanes0 = lax.broadcasted_iota(jnp.int32, (16,), 0)
            for qq in range(8):
                sl = pl.ds(qq * 16, 16)
                fill = (lanes0 + qq * 16) >= rem_vec
                sbuf_m[0, sl] = jnp.where(fill, 0, sbuf_m[0, sl])
                dbuf_m[0, sl] = jnp.where(fill, R, dbuf_m[0, sl])

            @pl.when(jnp.max(rem_vec) > 0)
            def _():
                pltpu.sync_copy(table_hbm.at[sbuf_m.at[0]], rows4.at[0])
                pltpu.sync_copy(rows4.at[0], acc.at[dbuf_m.at[0]], add=True)

            plsc.subcore_barrier()
            pltpu.sync_copy(acc.at[pl.ds(s * stripe, stripe)],
                            agg_hbm.at[pl.ds(lo + s * stripe, stripe)])

    cp = pltpu.CompilerParams()
    if "needs_layout_passes" in pltpu.CompilerParams.__dataclass_fields__:
        cp = dataclasses.replace(cp, needs_layout_passes=False)
    fn = pl.kernel(body, out_type=out_type,
                   mesh=mesh, scratch_types=scratch, compiler_params=cp,
                   interpret=interpret)
    return fn


def _rows_valid(i, shape):
    rows = lax.broadcasted_iota(jnp.int32, shape, 0) + i * BLKN
    return rows < N


def _msum(x, i):
    xm = jnp.where(_rows_valid(i, x.shape), x, 0.0)
    return jnp.sum(xm, axis=0), jnp.sum(xm * xm, axis=0)


def _bn_fold(s, n, g, be):
    mu = s[0] / n
    var = s[1] / n - mu * mu
    isd = g * lax.rsqrt(var + EPS)
    return isd, be - mu * isd


def _full(shape):
    nd = len(shape)
    return pl.BlockSpec(shape, lambda i, _n=nd: (0,) * _n)


def _rowblk(width=None, lead=None):
    if lead is None:
        if width is None:
            return pl.BlockSpec((BLKN,), lambda i: (i,))
        return pl.BlockSpec((BLKN, width), lambda i: (i, 0))
    return pl.BlockSpec((lead, BLKN, width), lambda i: (0, i, 0))


def _mm(a, b):
    return jnp.dot(a, b, preferred_element_type=jnp.float32)


def _k1a(agg_ref, x_ref, w1lt_ref, w1rt_ref, b1_ref,
         t_ref, inv_ref, st_ref):
    i = pl.program_id(0)
    cnt = agg_ref[:, FEAT:FEAT + 1]
    inv = 1.0 / jnp.maximum(cnt, 1.0)
    t = _mm(agg_ref[...] * inv, w1lt_ref[...]) \
        + _mm(x_ref[...], w1rt_ref[...]) + b1_ref[...]
    t_ref[...] = t
    inv_ref[...] = jnp.broadcast_to(inv, (BLKN, 8))
    s0, s1 = _msum(t, i)

    @pl.when(i == 0)
    def _():
        st_ref[...] = jnp.zeros_like(st_ref)
    st_ref[...] += jnp.stack([s0, s1])


def _k1b(t_ref, a_ref, cc_ref, w2lt_ref, w2rt_ref, b2_ref,
         h1l_ref, h1r_ref):
    h1 = jax.nn.relu(t_ref[...] * a_ref[...] + cc_ref[...])
    h1l_ref[...] = _mm(h1, w2lt_ref[...])
    h1r_ref[...] = _mm(h1, w2rt_ref[...]) + b2_ref[...]


def _k3a(agg_ref, inv_ref, h1r_ref, u_ref, st_ref):
    i = pl.program_id(0)
    u = agg_ref[...] * inv_ref[:, 0:1] + h1r_ref[...]
    u_ref[...] = u
    s0, s1 = _msum(u, i)

    @pl.when(i == 0)
    def _():
        st_ref[...] = jnp.zeros_like(st_ref)
    st_ref[...] += jnp.stack([s0, s1])


def _k3b(u_ref, a_ref, cc_ref, cwat_ref, cba_ref, dw1t_ref, db1_ref,
         h2_ref, hc1_ref, z1_ref, shc_ref, sz1_ref):
    i = pl.program_id(0)
    h2 = jax.nn.relu(u_ref[...] * a_ref[...] + cc_ref[...])
    h2_ref[...] = h2
    hc1 = jax.nn.relu(_mm(h2, cwat_ref[...]) + cba_ref[...])
    hc1_ref[...] = hc1

    @pl.when(i == 0)
    def _():
        shc_ref[...] = jnp.zeros_like(shc_ref)
        sz1_ref[...] = jnp.zeros_like(sz1_ref)

    s0, s1 = _msum(hc1, i)
    shc_ref[...] += jnp.stack([s0, s1])
    for k in range(K):
        z1k = jax.nn.relu(_mm(h2, dw1t_ref[k]) + db1_ref[k])
        z1_ref[k] = z1k
        s0, s1 = _msum(z1k, i)
        sz1_ref[k] += jnp.stack([s0, s1])


def _k3c(z1_ref, hc1_ref, ahc_ref, chc_ref, cwbt_ref, cbb_ref,
         az1_ref, cz1_ref, dw2t_ref, db2_ref,
         counts_ref, z2_ref, sz2_ref):
    i = pl.program_id(0)
    hc = hc1_ref[...] * ahc_ref[...] + chc_ref[...]
    counts_ref[...] = _mm(hc, cwbt_ref[...]) + cbb_ref[...]

    @pl.when(i == 0)
    def _():
        sz2_ref[...] = jnp.zeros_like(sz2_ref)

    for k in range(K):
        z1n = z1_ref[k] * az1_ref[k] + cz1_ref[k]
        z2k = jax.nn.relu(_mm(z1n, dw2t_ref[k]) + db2_ref[k])
        z2_ref[k] = z2k
        s0, s1 = _msum(z2k, i)
        sz2_ref[k] += jnp.stack([s0, s1])


def _k3d(z2_ref, az2_ref, cz2_ref, dw3t_ref, db3_ref, out_ref):
    for k in range(K):
        z2n = z2_ref[k] * az2_ref[k] + cz2_ref[k]
        out_ref[k] = jax.nn.sigmoid(_mm(z2n, dw3t_ref[k]) + db3_ref[k])


def _sds(*shape):
    return jax.ShapeDtypeStruct(shape, jnp.float32)


def kernel(x, edge_index, W1l, W1r, b1, g1, be1, W2l, W2r, b2, g2, be2,
           cWa, cba, cg, cbe, cWb, cbb, dW1, db1, dg1, dbe1, dW2, db2,
           dg2, dbe2, dW3, db3):
    src = jnp.pad(edge_index[0], (0, EP - E))
    dst = jnp.pad(edge_index[1], (0, EP - E), constant_values=1 << 30)
    xpad = jnp.pad(x, ((0, NP - N), (0, W1 - FEAT)))
    xpad = xpad.at[:, FEAT].set(1.0)

    agg1 = _make_sc_agg(W1, R2, 3)(src, dst, xpad)

    w1lt = jnp.pad(W1l, ((0, 0), (0, W1 - FEAT))).T
    w1rt = jnp.pad(W1r, ((0, 0), (0, W1 - FEAT))).T
    t, inv, st1 = pl.pallas_call(
        _k1a,
        grid=(NB,),
        in_specs=[_rowblk(W1), _rowblk(W1),
                  _full((W1, H)), _full((W1, H)), _full((1, H))],
        out_specs=[_rowblk(H), _rowblk(8), _full((2, H))],
        out_shape=[_sds(NP, H), _sds(NP, 8), _sds(2, H)],
    )(agg1, xpad, w1lt, w1rt, b1[None])

    a1, c1 = _bn_fold(st1, N, g1, be1)
    h1l, h1r = pl.pallas_call(
        _k1b,
        grid=(NB,),
        in_specs=[_rowblk(H), _full((1, H)), _full((1, H)),
                  _full((H, H)), _full((H, H)), _full((1, H))],
        out_specs=[_rowblk(H), _rowblk(H)],
        out_shape=[_sds(NP, H), _sds(NP, H)],
    )(t, a1[None], c1[None], W2l.T, W2r.T, b2[None])

    agg2 = _make_sc_agg(H, R2, 3)(src, dst, h1l)

    u, st2 = pl.pallas_call(
        _k3a,
        grid=(NB,),
        in_specs=[_rowblk(H), _rowblk(8), _rowblk(H)],
        out_specs=[_rowblk(H), _full((2, H))],
        out_shape=[_sds(NP, H), _sds(2, H)],
    )(agg2, inv, h1r)

    a2, c2 = _bn_fold(st2, N, g2, be2)
    dw1t = jnp.transpose(dW1, (0, 2, 1))
    h2, hc1, z1, shc, sz1 = pl.pallas_call(
        _k3b,
        grid=(NB,),
        in_specs=[_rowblk(H), _full((1, H)), _full((1, H)),
                  _full((H, 64)), _full((1, 64)),
                  _full((K, H, H)), _full((K, 1, H))],
        out_specs=[_rowblk(H), _rowblk(64), _rowblk(H, K),
                   _full((2, 64)), _full((K, 2, H))],
        out_shape=[_sds(NP, H), _sds(NP, 64), _sds(K, NP, H),
                   _sds(2, 64), _sds(K, 2, H)],
    )(u, a2[None], c2[None], cWa.T, cba[None], dw1t, db1[:, None, :])

    ahc, chc = _bn_fold(shc, N, cg, cbe)
    az1, cz1 = jax.vmap(_bn_fold, in_axes=(0, None, 0, 0))(sz1, N, dg1, dbe1)
    dw2t = jnp.transpose(dW2, (0, 2, 1))
    counts_p, z2, sz2 = pl.pallas_call(
        _k3c,
        grid=(NB,),
        in_specs=[_rowblk(H, K), _rowblk(64), _full((1, 64)), _full((1, 64)),
                  _full((64, K)), _full((1, K)),
                  _full((K, 1, H)), _full((K, 1, H)),
                  _full((K, H, 64)), _full((K, 1, 64))],
        out_specs=[_rowblk(K), _rowblk(64, K), _full((K, 2, 64))],
        out_shape=[_sds(NP, K), _sds(K, NP, 64), _sds(K, 2, 64)],
    )(z1, hc1, ahc[None], chc[None], cWb.T, cbb[None],
      az1[:, None, :], cz1[:, None, :], dw2t, db2[:, None, :])

    az2, cz2 = jax.vmap(_bn_fold, in_axes=(0, None, 0, 0))(sz2, N, dg2, dbe2)
    dw3t = jnp.transpose(dW3, (0, 2, 1))
    (sig,) = pl.pallas_call(
        _k3d,
        grid=(NB,),
        in_specs=[_rowblk(64, K), _full((K, 1, 64)), _full((K, 1, 64)),
                  _full((K, 64, 2 * M)), _full((K, 1, 2 * M))],
        out_specs=[_rowblk(2 * M, K)],
        out_shape=[_sds(K, NP, 2 * M)],
    )(z2, az2[:, None, :], cz2[:, None, :], dw3t, db3[:, None, :])

    counts = counts_p[:N]
    coords = sig[:, :N, :].reshape(K, N, M, 2)
    return (counts, coords)


# speedup vs baseline: 1.0316x; 1.0316x over previous
import dataclasses

import jax
import jax.numpy as jnp
from jax import lax
from jax.experimental import pallas as pl
from jax.experimental.pallas import tpu as pltpu
from jax.experimental.pallas import tpu_sc as plsc

N = 50000
NP = 53760
E = 800000
EP = 819200
FEAT = 17
W1 = 128
H = 128
K = 6
M = 12
EPS = 1e-5

BLK_E = 3200
NBLK_E = EP // 16 // BLK_E
SLICE_E = EP // 16
MROWS = 27

R2 = 8960
NR = 2

BLKN = 1680
NB = NP // BLKN

_SC_MESH = dict(core_axis_name="c", subcore_axis_name="s",
                num_cores=2, num_subcores=16)


def _make_sc_agg(width, rows_per_chunk, npass, interpret=False):
    R = rows_per_chunk
    stripe = R // 16
    mesh = plsc.VectorSubcoreMesh(**_SC_MESH)
    out_type = jax.ShapeDtypeStruct((NP, width), jnp.float32)
    scratch = (
        [pltpu.VMEM((BLK_E,), jnp.int32) for _ in range(4)]
        + [pltpu.VMEM((MROWS, 128), jnp.int32) for _ in range(2)]
        + [pltpu.VMEM((NR, 128, width), jnp.float32),
           pltpu.VMEM_SHARED((R + 16, width), jnp.float32)]
        + [pltpu.SemaphoreType.DMA] * (2 + 2 * NR)
    )

    def body(src_hbm, dst_hbm, table_hbm, agg_hbm,
             es0, ed0, es1, ed1, sbuf_m, dbuf_m, rows4, acc,
             sem_e0, sem_e1, *gssems):
        c = lax.axis_index("c")
        s = lax.axis_index("s")
        ebufs = ((es0, ed0), (es1, ed1))
        esems = (sem_e0, sem_e1)
        gsems = gssems[:NR]
        ssems = gssems[NR:]

        zero16 = jnp.zeros((16,), jnp.float32)

        def load_block(b_idx, which):
            base = s * SLICE_E + b_idx * BLK_E
            pltpu.async_copy(src_hbm.at[pl.ds(base, BLK_E)],
                             ebufs[which][0], esems[which])
            pltpu.async_copy(dst_hbm.at[pl.ds(base, BLK_E)],
                             ebufs[which][1], esems[which])

        def wait_block(which):
            pltpu.make_async_copy(src_hbm.at[pl.ds(0, BLK_E)],
                                  ebufs[which][0], esems[which]).wait()
            pltpu.make_async_copy(dst_hbm.at[pl.ds(0, BLK_E)],
                                  ebufs[which][1], esems[which]).wait()

        for p in range(npass):
            lo = (2 * p + c) * R

            @pl.loop(0, 128)
            def _(j):
                @pl.loop(0, width // 16)
                def _(q):
                    rows4[0, j, pl.ds(q * 16, 16)] = zero16
            for g in range(stripe // 128):
                pltpu.sync_copy(rows4.at[0],
                                acc.at[pl.ds(s * stripe + g * 128, 128)])
            rem_z = stripe % 128
            if rem_z:
                pltpu.sync_copy(
                    rows4.at[0].at[pl.ds(0, rem_z)],
                    acc.at[pl.ds(s * stripe + (stripe // 128) * 128, rem_z)])
            plsc.subcore_barrier()

            def scan(which, m_vec):
                sb, db = ebufs[which]

                def scan_vec(j, m):
                    dv = db[pl.ds(j * 16, 16)]
                    sv = sb[pl.ds(j * 16, 16)]
                    mask = (dv >= lo) & (dv < lo + R)
                    mi = jnp.where(mask, 1, 0).astype(jnp.int32)
                    csum = plsc.cumsum(mi)
                    pos = csum + m - 1
                    prow = lax.shift_right_logical(pos, 7)
                    pcol = lax.bitwise_and(pos, 127)
                    plsc.store_scatter(sbuf_m, [prow, pcol], sv, mask=mask)
                    plsc.store_scatter(dbuf_m, [prow, pcol], dv - lo,
                                       mask=mask)
                    return m + plsc.all_reduce_population_count(mask)

                return lax.fori_loop(0, BLK_E // 16, scan_vec, m_vec)

            def flush_full(m_vec):
                nf = lax.shift_right_logical(jnp.max(m_vec), 7)
                nq = lax.div(nf + (NR - 1), NR)

                def ring(g, _):
                    for b in range(NR):
                        r = g * NR + b
                        ok = r < nf

                        @pl.when(ok & (g > 0))
                        def _():
                            pltpu.make_async_copy(
                                rows4.at[b], acc.at[dbuf_m.at[r - NR]],
                                ssems[b]).wait()

                        @pl.when(ok)
                        def _():
                            pltpu.async_copy(table_hbm.at[sbuf_m.at[r]],
                                             rows4.at[b], gsems[b])
                    for b in range(NR):
                        r = g * NR + b
                        ok = r < nf

                        @pl.when(ok)
                        def _():
                            pltpu.make_async_copy(
                                table_hbm.at[sbuf_m.at[r]], rows4.at[b],
                                gsems[b]).wait()
                            pltpu.async_copy(rows4.at[b],
                                             acc.at[dbuf_m.at[r]],
                                             ssems[b], add=True)
                    return 0

                lax.fori_loop(0, nq, ring, 0)
                for b in range(NR):
                    @pl.when(nf > b)
                    def _():
                        pltpu.make_async_copy(rows4.at[b],
                                              acc.at[dbuf_m.at[0]],
                                              ssems[b]).wait()

                @pl.when(nf > 0)
                def _():
                    for qq in range(8):
                        sl = pl.ds(qq * 16, 16)
                        sbuf_m[0, sl] = sbuf_m[nf, sl]
                        dbuf_m[0, sl] = dbuf_m[nf, sl]
                return lax.bitwise_and(m_vec, 127)

            load_block(0, 0)

            def pair(q, m_vec):
                wait_block(0)
                load_block(2 * q + 1, 1)
                m_vec = flush_full(scan(0, m_vec))
                wait_block(1)

                @pl.when(q < NBLK_E // 2 - 1)
                def _():
                    load_block(2 * q + 2, 0)
                m_vec = flush_full(scan(1, m_vec))
                return m_vec

            m_vec = lax.fori_loop(0, NBLK_E // 2, pair,
                                  jnp.zeros((16,), jnp.int32))

            rem_vec = lax.bitwise_and(m_vec, 127)
            lanes0 = lax.broadcasted_iota(jnp.int32, (16,), 0)
            for qq in range(8):
                sl = pl.ds(qq * 16, 16)
                fill = (lanes0 + qq * 16) >= rem_vec
                sbuf_m[0, sl] = jnp.where(fill, 0, sbuf_m[0, sl])
                dbuf_m[0, sl] = jnp.where(fill, R, dbuf_m[0, sl])

            @pl.when(jnp.max(rem_vec) > 0)
            def _():
                pltpu.sync_copy(table_hbm.at[sbuf_m.at[0]], rows4.at[0])
                pltpu.sync_copy(rows4.at[0], acc.at[dbuf_m.at[0]], add=True)

            plsc.subcore_barrier()
            pltpu.sync_copy(acc.at[pl.ds(s * stripe, stripe)],
                            agg_hbm.at[pl.ds(lo + s * stripe, stripe)])

    cp = pltpu.CompilerParams()
    if "needs_layout_passes" in pltpu.CompilerParams.__dataclass_fields__:
        cp = dataclasses.replace(cp, needs_layout_passes=False)
    fn = pl.kernel(body, out_type=out_type,
                   mesh=mesh, scratch_types=scratch, compiler_params=cp,
                   interpret=interpret)
    return fn


def _rows_valid(i, shape):
    rows = lax.broadcasted_iota(jnp.int32, shape, 0) + i * BLKN
    return rows < N


def _msum(x, i):
    xm = jnp.where(_rows_valid(i, x.shape), x, 0.0)
    return jnp.sum(xm, axis=0), jnp.sum(xm * xm, axis=0)


def _bn_fold(s, n, g, be):
    mu = s[0] / n
    var = s[1] / n - mu * mu
    isd = g * lax.rsqrt(var + EPS)
    return isd, be - mu * isd


def _full(shape):
    nd = len(shape)
    return pl.BlockSpec(shape, lambda i, _n=nd: (0,) * _n)


def _rowblk(width=None, lead=None):
    if lead is None:
        if width is None:
            return pl.BlockSpec((BLKN,), lambda i: (i,))
        return pl.BlockSpec((BLKN, width), lambda i: (i, 0))
    return pl.BlockSpec((lead, BLKN, width), lambda i: (0, i, 0))


def _mm(a, b):
    return jnp.dot(a, b, preferred_element_type=jnp.float32)


def _k1a(agg_ref, x_ref, w1lt_ref, w1rt_ref, b1_ref,
         t_ref, inv_ref, st_ref):
    i = pl.program_id(0)
    cnt = agg_ref[:, FEAT:FEAT + 1]
    inv = 1.0 / jnp.maximum(cnt, 1.0)
    t = _mm(agg_ref[...] * inv, w1lt_ref[...]) \
        + _mm(x_ref[...], w1rt_ref[...]) + b1_ref[...]
    t_ref[...] = t
    inv_ref[...] = jnp.broadcast_to(inv, (BLKN, 8))
    s0, s1 = _msum(t, i)

    @pl.when(i == 0)
    def _():
        st_ref[...] = jnp.zeros_like(st_ref)
    st_ref[...] += jnp.stack([s0, s1])


def _k1b(t_ref, a_ref, cc_ref, w2lt_ref, w2rt_ref, b2_ref,
         h1l_ref, h1r_ref):
    h1 = jax.nn.relu(t_ref[...] * a_ref[...] + cc_ref[...])
    h1l_ref[...] = _mm(h1, w2lt_ref[...])
    h1r_ref[...] = _mm(h1, w2rt_ref[...]) + b2_ref[...]


def _k3a(agg_ref, inv_ref, h1r_ref, u_ref, st_ref):
    i = pl.program_id(0)
    u = agg_ref[...] * inv_ref[:, 0:1] + h1r_ref[...]
    u_ref[...] = u
    s0, s1 = _msum(u, i)

    @pl.when(i == 0)
    def _():
        st_ref[...] = jnp.zeros_like(st_ref)
    st_ref[...] += jnp.stack([s0, s1])


def _k3b(u_ref, a_ref, cc_ref, cwat_ref, cba_ref, dw1t_ref, db1_ref,
         h2_ref, hc1_ref, shc_ref, sz1_ref):
    i = pl.program_id(0)
    h2 = jax.nn.relu(u_ref[...] * a_ref[...] + cc_ref[...])
    h2_ref[...] = h2
    hc1 = jax.nn.relu(_mm(h2, cwat_ref[...]) + cba_ref[...])
    hc1_ref[...] = hc1

    @pl.when(i == 0)
    def _():
        shc_ref[...] = jnp.zeros_like(shc_ref)
        sz1_ref[...] = jnp.zeros_like(sz1_ref)

    s0, s1 = _msum(hc1, i)
    shc_ref[...] += jnp.stack([s0, s1])
    for k in range(K):
        z1k = jax.nn.relu(_mm(h2, dw1t_ref[k]) + db1_ref[k])
        s0, s1 = _msum(z1k, i)
        sz1_ref[k] += jnp.stack([s0, s1])


def _k3c(h2_ref, hc1_ref, ahc_ref, chc_ref, cwbt_ref, cbb_ref,
         dw1t_ref, db1_ref, az1_ref, cz1_ref, dw2t_ref, db2_ref,
         counts_ref, z2_ref, sz2_ref):
    i = pl.program_id(0)
    hc = hc1_ref[...] * ahc_ref[...] + chc_ref[...]
    counts_ref[...] = _mm(hc, cwbt_ref[...]) + cbb_ref[...]

    @pl.when(i == 0)
    def _():
        sz2_ref[...] = jnp.zeros_like(sz2_ref)

    for k in range(K):
        z1k = jax.nn.relu(_mm(h2_ref[...], dw1t_ref[k]) + db1_ref[k])
        z1n = z1k * az1_ref[k] + cz1_ref[k]
        z2k = jax.nn.relu(_mm(z1n, dw2t_ref[k]) + db2_ref[k])
        z2_ref[k] = z2k
        s0, s1 = _msum(z2k, i)
        sz2_ref[k] += jnp.stack([s0, s1])


def _k3d(z2_ref, az2_ref, cz2_ref, dw3t_ref, db3_ref, out_ref):
    for k in range(K):
        z2n = z2_ref[k] * az2_ref[k] + cz2_ref[k]
        out_ref[k] = jax.nn.sigmoid(_mm(z2n, dw3t_ref[k]) + db3_ref[k])


def _sds(*shape):
    return jax.ShapeDtypeStruct(shape, jnp.float32)


def kernel(x, edge_index, W1l, W1r, b1, g1, be1, W2l, W2r, b2, g2, be2,
           cWa, cba, cg, cbe, cWb, cbb, dW1, db1, dg1, dbe1, dW2, db2,
           dg2, dbe2, dW3, db3):
    src = jnp.pad(edge_index[0], (0, EP - E))
    dst = jnp.pad(edge_index[1], (0, EP - E), constant_values=1 << 30)
    xpad = jnp.pad(x, ((0, NP - N), (0, W1 - FEAT)))
    xpad = xpad.at[:, FEAT].set(1.0)

    agg1 = _make_sc_agg(W1, R2, 3)(src, dst, xpad)

    w1lt = jnp.pad(W1l, ((0, 0), (0, W1 - FEAT))).T
    w1rt = jnp.pad(W1r, ((0, 0), (0, W1 - FEAT))).T
    t, inv, st1 = pl.pallas_call(
        _k1a,
        grid=(NB,),
        in_specs=[_rowblk(W1), _rowblk(W1),
                  _full((W1, H)), _full((W1, H)), _full((1, H))],
        out_specs=[_rowblk(H), _rowblk(8), _full((2, H))],
        out_shape=[_sds(NP, H), _sds(NP, 8), _sds(2, H)],
    )(agg1, xpad, w1lt, w1rt, b1[None])

    a1, c1 = _bn_fold(st1, N, g1, be1)
    h1l, h1r = pl.pallas_call(
        _k1b,
        grid=(NB,),
        in_specs=[_rowblk(H), _full((1, H)), _full((1, H)),
                  _full((H, H)), _full((H, H)), _full((1, H))],
        out_specs=[_rowblk(H), _rowblk(H)],
        out_shape=[_sds(NP, H), _sds(NP, H)],
    )(t, a1[None], c1[None], W2l.T, W2r.T, b2[None])

    agg2 = _make_sc_agg(H, R2, 3)(src, dst, h1l)

    u, st2 = pl.pallas_call(
        _k3a,
        grid=(NB,),
        in_specs=[_rowblk(H), _rowblk(8), _rowblk(H)],
        out_specs=[_rowblk(H), _full((2, H))],
        out_shape=[_sds(NP, H), _sds(2, H)],
    )(agg2, inv, h1r)

    a2, c2 = _bn_fold(st2, N, g2, be2)
    dw1t = jnp.transpose(dW1, (0, 2, 1))
    h2, hc1, shc, sz1 = pl.pallas_call(
        _k3b,
        grid=(NB,),
        in_specs=[_rowblk(H), _full((1, H)), _full((1, H)),
                  _full((H, 64)), _full((1, 64)),
                  _full((K, H, H)), _full((K, 1, H))],
        out_specs=[_rowblk(H), _rowblk(64),
                   _full((2, 64)), _full((K, 2, H))],
        out_shape=[_sds(NP, H), _sds(NP, 64),
                   _sds(2, 64), _sds(K, 2, H)],
    )(u, a2[None], c2[None], cWa.T, cba[None], dw1t, db1[:, None, :])

    ahc, chc = _bn_fold(shc, N, cg, cbe)
    az1, cz1 = jax.vmap(_bn_fold, in_axes=(0, None, 0, 0))(sz1, N, dg1, dbe1)
    dw2t = jnp.transpose(dW2, (0, 2, 1))
    counts_p, z2, sz2 = pl.pallas_call(
        _k3c,
        grid=(NB,),
        in_specs=[_rowblk(H), _rowblk(64), _full((1, 64)), _full((1, 64)),
                  _full((64, K)), _full((1, K)),
                  _full((K, H, H)), _full((K, 1, H)),
                  _full((K, 1, H)), _full((K, 1, H)),
                  _full((K, H, 64)), _full((K, 1, 64))],
        out_specs=[_rowblk(K), _rowblk(64, K), _full((K, 2, 64))],
        out_shape=[_sds(NP, K), _sds(K, NP, 64), _sds(K, 2, 64)],
    )(h2, hc1, ahc[None], chc[None], cWb.T, cbb[None],
      dw1t, db1[:, None, :], az1[:, None, :], cz1[:, None, :],
      dw2t, db2[:, None, :])

    az2, cz2 = jax.vmap(_bn_fold, in_axes=(0, None, 0, 0))(sz2, N, dg2, dbe2)
    dw3t = jnp.transpose(dW3, (0, 2, 1))
    (sig,) = pl.pallas_call(
        _k3d,
        grid=(NB,),
        in_specs=[_rowblk(64, K), _full((K, 1, 64)), _full((K, 1, 64)),
                  _full((K, 64, 2 * M)), _full((K, 1, 2 * M))],
        out_specs=[_rowblk(2 * M, K)],
        out_shape=[_sds(K, NP, 2 * M)],
    )(z2, az2[:, None, :], cz2[:, None, :], dw3t, db3[:, None, :])

    counts = counts_p[:N]
    coords = sig[:, :N, :].reshape(K, N, M, 2)
    return (counts, coords)


# speedup vs baseline: 1.1035x; 1.0697x over previous
import dataclasses

import jax
import jax.numpy as jnp
from jax import lax
from jax.experimental import pallas as pl
from jax.experimental.pallas import tpu as pltpu
from jax.experimental.pallas import tpu_sc as plsc

N = 50000
NP = 53760
E = 800000
EP = 819200
FEAT = 17
W1 = 128
H = 128
K = 6
M = 12
EPS = 1e-5

BLK_E = 3200
NBLK_E = EP // 16 // BLK_E
SLICE_E = EP // 16
MROWS = 27

R2 = 8960
NR = 2

BLKN = 1680
NB = NP // BLKN

_SC_MESH = dict(core_axis_name="c", subcore_axis_name="s",
                num_cores=2, num_subcores=16)


def _make_sc_agg(width, rows_per_chunk, npass, interpret=False):
    R = rows_per_chunk
    stripe = R // 16
    mesh = plsc.VectorSubcoreMesh(**_SC_MESH)
    out_type = jax.ShapeDtypeStruct((NP, width), jnp.float32)
    scratch = (
        [pltpu.VMEM((BLK_E,), jnp.int32) for _ in range(4)]
        + [pltpu.VMEM((MROWS, 128), jnp.int32) for _ in range(2)]
        + [pltpu.VMEM((NR, 128, width), jnp.float32),
           pltpu.VMEM_SHARED((R + 16, width), jnp.float32)]
        + [pltpu.SemaphoreType.DMA] * (2 + 2 * NR)
    )

    def body(src_hbm, dst_hbm, table_hbm, agg_hbm,
             es0, ed0, es1, ed1, sbuf_m, dbuf_m, rows4, acc,
             sem_e0, sem_e1, *gssems):
        c = lax.axis_index("c")
        s = lax.axis_index("s")
        ebufs = ((es0, ed0), (es1, ed1))
        esems = (sem_e0, sem_e1)
        gsems = gssems[:NR]
        ssems = gssems[NR:]

        zero16 = jnp.zeros((16,), jnp.float32)

        def load_block(b_idx, which):
            base = s * SLICE_E + b_idx * BLK_E
            pltpu.async_copy(src_hbm.at[pl.ds(base, BLK_E)],
                             ebufs[which][0], esems[which])
            pltpu.async_copy(dst_hbm.at[pl.ds(base, BLK_E)],
                             ebufs[which][1], esems[which])

        def wait_block(which):
            pltpu.make_async_copy(src_hbm.at[pl.ds(0, BLK_E)],
                                  ebufs[which][0], esems[which]).wait()
            pltpu.make_async_copy(dst_hbm.at[pl.ds(0, BLK_E)],
                                  ebufs[which][1], esems[which]).wait()

        for p in range(npass):
            lo = (2 * p + c) * R

            @pl.loop(0, 128)
            def _(j):
                @pl.loop(0, width // 16)
                def _(q):
                    rows4[0, j, pl.ds(q * 16, 16)] = zero16
            for g in range(stripe // 128):
                pltpu.sync_copy(rows4.at[0],
                                acc.at[pl.ds(s * stripe + g * 128, 128)])
            rem_z = stripe % 128
            if rem_z:
                pltpu.sync_copy(
                    rows4.at[0].at[pl.ds(0, rem_z)],
                    acc.at[pl.ds(s * stripe + (stripe // 128) * 128, rem_z)])
            plsc.subcore_barrier()

            def scan(which, m_vec):
                sb, db = ebufs[which]

                def scan_vec(j, m):
                    dv = db[pl.ds(j * 16, 16)]
                    sv = sb[pl.ds(j * 16, 16)]
                    mask = (dv >= lo) & (dv < lo + R)
                    mi = jnp.where(mask, 1, 0).astype(jnp.int32)
                    csum = plsc.cumsum(mi)
                    pos = csum + m - 1
                    prow = lax.shift_right_logical(pos, 7)
                    pcol = lax.bitwise_and(pos, 127)
                    plsc.store_scatter(sbuf_m, [prow, pcol], sv, mask=mask)
                    plsc.store_scatter(dbuf_m, [prow, pcol], dv - lo,
                                       mask=mask)
                    return m + plsc.all_reduce_population_count(mask)

                return lax.fori_loop(0, BLK_E // 16, scan_vec, m_vec)

            def flush_full(m_vec):
                nf = lax.shift_right_logical(jnp.max(m_vec), 7)
                nq = lax.div(nf + (NR - 1), NR)

                def ring(g, _):
                    for b in range(NR):
                        r = g * NR + b
                        ok = r < nf

                        @pl.when(ok & (g > 0))
                        def _():
                            pltpu.make_async_copy(
                                rows4.at[b], acc.at[dbuf_m.at[r - NR]],
                                ssems[b]).wait()

                        @pl.when(ok)
                        def _():
                            pltpu.async_copy(table_hbm.at[sbuf_m.at[r]],
                                             rows4.at[b], gsems[b])
                    for b in range(NR):
                        r = g * NR + b
                        ok = r < nf

                        @pl.when(ok)
                        def _():
                            pltpu.make_async_copy(
                                table_hbm.at[sbuf_m.at[r]], rows4.at[b],
                                gsems[b]).wait()
                            pltpu.async_copy(rows4.at[b],
                                             acc.at[dbuf_m.at[r]],
                                             ssems[b], add=True)
                    return 0

                lax.fori_loop(0, nq, ring, 0)
                for b in range(NR):
                    @pl.when(nf > b)
                    def _():
                        pltpu.make_async_copy(rows4.at[b],
                                              acc.at[dbuf_m.at[0]],
                                              ssems[b]).wait()

                @pl.when(nf > 0)
                def _():
                    for qq in range(8):
                        sl = pl.ds(qq * 16, 16)
                        sbuf_m[0, sl] = sbuf_m[nf, sl]
                        dbuf_m[0, sl] = dbuf_m[nf, sl]
                return lax.bitwise_and(m_vec, 127)

            load_block(0, 0)

            def pair(q, m_vec):
                wait_block(0)
                load_block(2 * q + 1, 1)
                m_vec = flush_full(scan(0, m_vec))
                wait_block(1)

                @pl.when(q < NBLK_E // 2 - 1)
                def _():
                    load_block(2 * q + 2, 0)
                m_vec = flush_full(scan(1, m_vec))
                return m_vec

            m_vec = lax.fori_loop(0, NBLK_E // 2, pair,
                                  jnp.zeros((16,), jnp.int32))

            rem_vec = lax.bitwise_and(m_vec, 127)
            lanes0 = lax.broadcasted_iota(jnp.int32, (16,), 0)
            for qq in range(8):
                sl = pl.ds(qq * 16, 16)
                fill = (lanes0 + qq * 16) >= rem_vec
                sbuf_m[0, sl] = jnp.where(fill, 0, sbuf_m[0, sl])
                dbuf_m[0, sl] = jnp.where(fill, R, dbuf_m[0, sl])

            @pl.when(jnp.max(rem_vec) > 0)
            def _():
                pltpu.sync_copy(table_hbm.at[sbuf_m.at[0]], rows4.at[0])
                pltpu.sync_copy(rows4.at[0], acc.at[dbuf_m.at[0]], add=True)

            plsc.subcore_barrier()
            pltpu.sync_copy(acc.at[pl.ds(s * stripe, stripe)],
                            agg_hbm.at[pl.ds(lo + s * stripe, stripe)])

    cp = pltpu.CompilerParams()
    if "needs_layout_passes" in pltpu.CompilerParams.__dataclass_fields__:
        cp = dataclasses.replace(cp, needs_layout_passes=False)
    fn = pl.kernel(body, out_type=out_type,
                   mesh=mesh, scratch_types=scratch, compiler_params=cp,
                   interpret=interpret)
    return fn


def _rows_valid(i, shape):
    rows = lax.broadcasted_iota(jnp.int32, shape, 0) + i * BLKN
    return rows < N


def _msum(x, i):
    xm = jnp.where(_rows_valid(i, x.shape), x, 0.0)
    return jnp.sum(xm, axis=0), jnp.sum(xm * xm, axis=0)


def _bn_fold(s, n, g, be):
    mu = s[0] / n
    var = s[1] / n - mu * mu
    isd = g * lax.rsqrt(var + EPS)
    return isd, be - mu * isd


def _full(shape):
    nd = len(shape)
    return pl.BlockSpec(shape, lambda i, _n=nd: (0,) * _n)


def _rowblk(width=None, lead=None):
    if lead is None:
        if width is None:
            return pl.BlockSpec((BLKN,), lambda i: (i,))
        return pl.BlockSpec((BLKN, width), lambda i: (i, 0))
    return pl.BlockSpec((lead, BLKN, width), lambda i: (0, i, 0))


def _mm(a, b):
    return jnp.dot(a, b, preferred_element_type=jnp.float32)


def _k1a(agg_ref, x_ref, w1lt_ref, w1rt_ref, b1_ref,
         t_ref, inv_ref, st_ref):
    i = pl.program_id(0)
    cnt = agg_ref[:, FEAT:FEAT + 1]
    inv = 1.0 / jnp.maximum(cnt, 1.0)
    t = _mm(agg_ref[...] * inv, w1lt_ref[...]) \
        + _mm(x_ref[...], w1rt_ref[...]) + b1_ref[...]
    t_ref[...] = t
    inv_ref[...] = jnp.broadcast_to(inv, (BLKN, 8))
    s0, s1 = _msum(t, i)

    @pl.when(i == 0)
    def _():
        st_ref[...] = jnp.zeros_like(st_ref)
    st_ref[...] += jnp.stack([s0, s1])


def _k1b(t_ref, a_ref, cc_ref, w2lt_ref, w2rt_ref, b2_ref,
         h1l_ref, h1r_ref):
    h1 = jax.nn.relu(t_ref[...] * a_ref[...] + cc_ref[...])
    h1l_ref[...] = _mm(h1, w2lt_ref[...])
    h1r_ref[...] = _mm(h1, w2rt_ref[...]) + b2_ref[...]


def _k3a(agg_ref, inv_ref, h1r_ref, u_ref, st_ref):
    i = pl.program_id(0)
    u = agg_ref[...] * inv_ref[:, 0:1] + h1r_ref[...]
    u_ref[...] = u
    s0, s1 = _msum(u, i)

    @pl.when(i == 0)
    def _():
        st_ref[...] = jnp.zeros_like(st_ref)
    st_ref[...] += jnp.stack([s0, s1])


def _k3b(u_ref, a_ref, cc_ref, cwat_ref, cba_ref, dw1t_ref, db1_ref,
         h2_ref, hc1_ref, shc_ref, sz1_ref):
    i = pl.program_id(0)
    h2 = jax.nn.relu(u_ref[...] * a_ref[...] + cc_ref[...])
    h2_ref[...] = h2
    hc1 = jax.nn.relu(_mm(h2, cwat_ref[...]) + cba_ref[...])
    hc1_ref[...] = hc1

    @pl.when(i == 0)
    def _():
        shc_ref[...] = jnp.zeros_like(shc_ref)
        sz1_ref[...] = jnp.zeros_like(sz1_ref)

    s0, s1 = _msum(hc1, i)
    shc_ref[...] += jnp.stack([s0, s1])
    for k in range(K):
        z1k = jax.nn.relu(_mm(h2, dw1t_ref[k]) + db1_ref[k])
        s0, s1 = _msum(z1k, i)
        sz1_ref[k] += jnp.stack([s0, s1])


def _k3c(h2_ref, hc1_ref, ahc_ref, chc_ref, cwbt_ref, cbb_ref,
         dw1t_ref, db1_ref, az1_ref, cz1_ref, dw2t_ref, db2_ref,
         counts_ref, z2_ref, sz2_ref):
    i = pl.program_id(0)
    hc = hc1_ref[...] * ahc_ref[...] + chc_ref[...]
    counts_ref[...] = _mm(hc, cwbt_ref[...]) + cbb_ref[...]

    @pl.when(i == 0)
    def _():
        sz2_ref[...] = jnp.zeros_like(sz2_ref)

    for k in range(K):
        z1k = jax.nn.relu(_mm(h2_ref[...], dw1t_ref[k]) + db1_ref[k])
        z1n = z1k * az1_ref[k] + cz1_ref[k]
        z2k = jax.nn.relu(_mm(z1n, dw2t_ref[k]) + db2_ref[k])
        z2_ref[k] = z2k
        s0, s1 = _msum(z2k, i)
        sz2_ref[k] += jnp.stack([s0, s1])


def _k3d(z2_ref, az2_ref, cz2_ref, dw3t_ref, db3_ref, out_ref):
    for k in range(K):
        z2n = z2_ref[k] * az2_ref[k] + cz2_ref[k]
        out_ref[k] = jax.nn.sigmoid(_mm(z2n, dw3t_ref[k]) + db3_ref[k])


def _kpad(x_ref, out_ref):
    xp = jnp.pad(x_ref[...], ((0, 0), (0, W1 - FEAT)))
    col = lax.broadcasted_iota(jnp.int32, xp.shape, 1)
    out_ref[...] = jnp.where(col == FEAT, 1.0, xp)


def _sds(*shape):
    return jax.ShapeDtypeStruct(shape, jnp.float32)


def kernel(x, edge_index, W1l, W1r, b1, g1, be1, W2l, W2r, b2, g2, be2,
           cWa, cba, cg, cbe, cWb, cbb, dW1, db1, dg1, dbe1, dW2, db2,
           dg2, dbe2, dW3, db3):
    src = jnp.pad(edge_index[0], (0, EP - E))
    dst = jnp.pad(edge_index[1], (0, EP - E), constant_values=1 << 30)
    xpad = pl.pallas_call(
        _kpad,
        grid=(N // 2000,),
        in_specs=[pl.BlockSpec((2000, FEAT), lambda i: (i, 0))],
        out_specs=pl.BlockSpec((2000, W1), lambda i: (i, 0)),
        out_shape=_sds(NP, W1),
    )(x)

    agg1 = _make_sc_agg(W1, R2, 3)(src, dst, xpad)

    w1lt = jnp.pad(W1l, ((0, 0), (0, W1 - FEAT))).T
    w1rt = jnp.pad(W1r, ((0, 0), (0, W1 - FEAT))).T
    t, inv, st1 = pl.pallas_call(
        _k1a,
        grid=(NB,),
        in_specs=[_rowblk(W1), _rowblk(W1),
                  _full((W1, H)), _full((W1, H)), _full((1, H))],
        out_specs=[_rowblk(H), _rowblk(8), _full((2, H))],
        out_shape=[_sds(NP, H), _sds(NP, 8), _sds(2, H)],
    )(agg1, xpad, w1lt, w1rt, b1[None])

    a1, c1 = _bn_fold(st1, N, g1, be1)
    h1l, h1r = pl.pallas_call(
        _k1b,
        grid=(NB,),
        in_specs=[_rowblk(H), _full((1, H)), _full((1, H)),
                  _full((H, H)), _full((H, H)), _full((1, H))],
        out_specs=[_rowblk(H), _rowblk(H)],
        out_shape=[_sds(NP, H), _sds(NP, H)],
    )(t, a1[None], c1[None], W2l.T, W2r.T, b2[None])

    agg2 = _make_sc_agg(H, R2, 3)(src, dst, h1l)

    u, st2 = pl.pallas_call(
        _k3a,
        grid=(NB,),
        in_specs=[_rowblk(H), _rowblk(8), _rowblk(H)],
        out_specs=[_rowblk(H), _full((2, H))],
        out_shape=[_sds(NP, H), _sds(2, H)],
    )(agg2, inv, h1r)

    a2, c2 = _bn_fold(st2, N, g2, be2)
    dw1t = jnp.transpose(dW1, (0, 2, 1))
    h2, hc1, shc, sz1 = pl.pallas_call(
        _k3b,
        grid=(NB,),
        in_specs=[_rowblk(H), _full((1, H)), _full((1, H)),
                  _full((H, 64)), _full((1, 64)),
                  _full((K, H, H)), _full((K, 1, H))],
        out_specs=[_rowblk(H), _rowblk(64),
                   _full((2, 64)), _full((K, 2, H))],
        out_shape=[_sds(NP, H), _sds(NP, 64),
                   _sds(2, 64), _sds(K, 2, H)],
    )(u, a2[None], c2[None], cWa.T, cba[None], dw1t, db1[:, None, :])

    ahc, chc = _bn_fold(shc, N, cg, cbe)
    az1, cz1 = jax.vmap(_bn_fold, in_axes=(0, None, 0, 0))(sz1, N, dg1, dbe1)
    dw2t = jnp.transpose(dW2, (0, 2, 1))
    counts_p, z2, sz2 = pl.pallas_call(
        _k3c,
        grid=(NB,),
        in_specs=[_rowblk(H), _rowblk(64), _full((1, 64)), _full((1, 64)),
                  _full((64, K)), _full((1, K)),
                  _full((K, H, H)), _full((K, 1, H)),
                  _full((K, 1, H)), _full((K, 1, H)),
                  _full((K, H, 64)), _full((K, 1, 64))],
        out_specs=[_rowblk(K), _rowblk(64, K), _full((K, 2, 64))],
        out_shape=[_sds(NP, K), _sds(K, NP, 64), _sds(K, 2, 64)],
    )(h2, hc1, ahc[None], chc[None], cWb.T, cbb[None],
      dw1t, db1[:, None, :], az1[:, None, :], cz1[:, None, :],
      dw2t, db2[:, None, :])

    az2, cz2 = jax.vmap(_bn_fold, in_axes=(0, None, 0, 0))(sz2, N, dg2, dbe2)
    dw3t = jnp.transpose(dW3, (0, 2, 1))
    (sig,) = pl.pallas_call(
        _k3d,
        grid=(N // 2000,),
        in_specs=[pl.BlockSpec((K, 2000, 64), lambda i: (0, i, 0)),
                  _full((K, 1, 64)), _full((K, 1, 64)),
                  _full((K, 64, 2 * M)), _full((K, 1, 2 * M))],
        out_specs=[pl.BlockSpec((K, 2000, 2 * M), lambda i: (0, i, 0))],
        out_shape=[_sds(K, N, 2 * M)],
    )(z2, az2[:, None, :], cz2[:, None, :], dw3t, db3[:, None, :])

    counts = counts_p[:N]
    coords = sig.reshape(K, N, M, 2)
    return (counts, coords)
